# baseline (speedup 1.0000x reference)
import jax
import jax.numpy as jnp
from jax import lax
from jax.experimental import pallas as pl
from jax.experimental.pallas import tpu as pltpu
from jax.experimental.pallas import tpu_sc as plsc

_VOCAB = 100000
_D = 300
_B = 4096
_L = 200
_C = 2

_NC = 2
_NS = 16
_NW = _NC * _NS
_ROWS_W = _B // _NW
_GROUPS = _ROWS_W // 16
_KCH = _L // 8
_GPAIR = _GROUPS // 2
_NCH = _GPAIR * _KCH
_CH = 256
_NSLOT = 8

_VCH = 2048
_NFULL = _VOCAB // _VCH
_VTAIL = _VOCAB - _NFULL * _VCH
_NBUF = 6


def _pack(p, fcb, n):
    p = (p + fcb) * (1.0 / _L)
    u = lax.bitcast_convert_type(
        p.astype(jnp.bfloat16), jnp.uint16).astype(jnp.uint32)
    return lax.bitcast_convert_type((u[0] << 16) | u[1], jnp.int32)


def _proj_body(embT_hbm, fcw_ref, fcb_ref, out_ref, buf, tbuf, tsem, sems):
    def _copy(c, s):
        return pltpu.make_async_copy(
            embT_hbm.at[:, pl.ds(c * _VCH, _VCH)], buf.at[s], sems.at[s])

    for s in range(_NBUF):
        _copy(s, s).start()
    pltpu.make_async_copy(
        embT_hbm.at[:, pl.ds(_NFULL * _VCH, _VTAIL)], tbuf, tsem).start()

    fcw = fcw_ref[...]
    fcb = fcb_ref[...]

    def _step(c, carry):
        s = lax.rem(c, _NBUF)
        _copy(c, s).wait()
        p = lax.dot_general(fcw, buf[s], (((1,), (0,)), ((), ())),
                            preferred_element_type=jnp.float32)
        out_ref[pl.ds(c * _VCH, _VCH)] = _pack(p, fcb, _VCH)

        @pl.when(c + _NBUF < _NFULL)
        def _():
            _copy(c + _NBUF, s).start()
        return carry

    lax.fori_loop(0, _NFULL, _step, 0)
    pltpu.make_async_copy(
        embT_hbm.at[:, pl.ds(_NFULL * _VCH, _VTAIL)], tbuf, tsem).wait()
    p = lax.dot_general(fcw, tbuf[...], (((1,), (0,)), ((), ())),
                        preferred_element_type=jnp.float32)
    out_ref[pl.ds(_NFULL * _VCH, _VTAIL)] = _pack(p, fcb, _VTAIL)


def _project(embT, fcw, fcb):
    return pl.pallas_call(
        _proj_body,
        in_specs=[
            pl.BlockSpec(memory_space=pl.ANY),
            pl.BlockSpec((_C, _D), lambda: (0, 0)),
            pl.BlockSpec((_C, 1), lambda: (0, 0)),
        ],
        out_specs=pl.BlockSpec((_VOCAB,), lambda: (0,)),
        out_shape=jax.ShapeDtypeStruct((_VOCAB,), jnp.int32),
        scratch_shapes=[
            pltpu.VMEM((_NBUF, _D, _VCH), jnp.float32),
            pltpu.VMEM((_D, _VTAIL), jnp.float32),
            pltpu.SemaphoreType.DMA,
            pltpu.SemaphoreType.DMA((_NBUF,)),
        ],
    )(embT, fcw, fcb)


def _sc_body(p_hbm, idx_hbm, lane_hbm, out_hbm,
             idx_v, rows_v, lane_v, out_v, *sems):
    wid = lax.axis_index("s") * _NC + lax.axis_index("c")
    pltpu.sync_copy(idx_hbm.at[wid], idx_v)
    pltpu.sync_copy(lane_hbm, lane_v)

    lane16 = [lane_v[v] for v in range(16)]
    himask = lane_v[16]
    sh16 = lane_v[17]

    def _gather(j, s):
        return pltpu.make_async_copy(
            p_hbm.at[idx_v.at[j]], rows_v.at[s], sems[s])

    for s in range(_NSLOT):
        _gather(s, s).start()

    def _reduce(j, s, start_next):
        _gather(j, s).wait()
        row = rows_v.at[s]
        gp = j // _KCH
        k = j - gp * _KCH
        for h in range(2):
            uv = plsc.load_gather(row, [lane16[h * 8]])
            a0 = plsc.bitcast(uv & himask, jnp.float32)
            a1 = plsc.bitcast(lax.shift_left(uv, sh16), jnp.float32)
            for v in range(h * 8 + 1, h * 8 + 8):
                uv = plsc.load_gather(row, [lane16[v]])
                a0 = a0 + plsc.bitcast(uv & himask, jnp.float32)
                a1 = a1 + plsc.bitcast(
                    lax.shift_left(uv, sh16), jnp.float32)
            o0 = pl.ds((2 * gp + h) * 16, 16)
            o1 = pl.ds(128 + (2 * gp + h) * 16, 16)

            @pl.when(k == 0)
            def _():
                out_v[o0] = a0
                out_v[o1] = a1

            @pl.when(k != 0)
            def _():
                out_v[o0] = out_v[o0] + a0
                out_v[o1] = out_v[o1] + a1

        if start_next:
            @pl.when(j + _NSLOT < _NCH)
            def _():
                _gather(j + _NSLOT, s).start()

    def _outer(j0, c):
        for s in range(_NSLOT):
            _reduce(j0 * _NSLOT + s, s, True)
        return c

    lax.fori_loop(0, _NCH // _NSLOT, _outer, 0)
    for s in range(_NCH - (_NCH // _NSLOT) * _NSLOT):
        _reduce((_NCH // _NSLOT) * _NSLOT + s, s, False)
    pltpu.sync_copy(out_v, out_hbm.at[wid])


_lookup = pl.kernel(
    _sc_body,
    out_type=jax.ShapeDtypeStruct((_NW, _C * _ROWS_W), jnp.float32),
    mesh=plsc.VectorSubcoreMesh(core_axis_name="c", subcore_axis_name="s"),
    compiler_params=pltpu.CompilerParams(
        needs_layout_passes=False, use_tc_tiling_on_sc=False),
    scratch_types=[
        pltpu.VMEM((_NCH, _CH), jnp.int32),
        pltpu.VMEM((_NSLOT, _CH), jnp.int32),
        pltpu.VMEM((18, 16), jnp.int32),
        pltpu.VMEM((_C * _ROWS_W,), jnp.float32),
    ] + [pltpu.SemaphoreType.DMA] * _NSLOT,
)

_LANE_TAB = [[v * 16 + i for i in range(16)] for v in range(16)]
_LANE_TAB.append([-65536] * 16)
_LANE_TAB.append([16] * 16)


def kernel(text, emb_table, fc_w, fc_b):
    embT = emb_table.T
    ptab = _project(embT, fc_w.astype(jnp.float32),
                    fc_b.astype(jnp.float32).reshape(_C, 1))
    textT = text.astype(jnp.int32).T
    u = textT.reshape(_KCH, 8, _NW, _GPAIR, 2, 16)
    idx = u.transpose(2, 3, 0, 4, 1, 5).reshape(_NW, _NCH, _CH)
    lane_tab = jnp.asarray(_LANE_TAB, jnp.int32)
    out32 = _lookup(ptab, idx, lane_tab)
    return out32.reshape(_NW, _C, _ROWS_W).transpose(0, 2, 1).reshape(_B, _C)


# speedup vs baseline: 1.5276x; 1.5276x over previous
import jax
import jax.numpy as jnp
from jax import lax
from jax.experimental import pallas as pl
from jax.experimental.pallas import tpu as pltpu
from jax.experimental.pallas import tpu_sc as plsc

_VOCAB = 100000
_D = 300
_B = 4096
_L = 200
_C = 2

_NC = 2
_NS = 16
_NW = _NC * _NS
_ROWS_W = _B // _NW
_GROUPS = _ROWS_W // 16
_KCH = _L // 8
_NCH = _GROUPS * _KCH
_CH = 128
_NSLOT = 8

_VCH = 2048
_NFULL = _VOCAB // _VCH
_VTAIL = _VOCAB - _NFULL * _VCH
_NBUF = 6


def _pack(p, fcb, n):
    p = (p + fcb) * (1.0 / _L)
    u = lax.bitcast_convert_type(
        p.astype(jnp.bfloat16), jnp.uint16).astype(jnp.uint32)
    return lax.bitcast_convert_type((u[0] << 16) | u[1], jnp.int32)


def _proj_body(embT_hbm, fcw_ref, fcb_ref, out_ref, buf, tbuf, tsem, sems):
    def _copy(c, s):
        return pltpu.make_async_copy(
            embT_hbm.at[:, pl.ds(c * _VCH, _VCH)], buf.at[s], sems.at[s])

    for s in range(_NBUF):
        _copy(s, s).start()
    pltpu.make_async_copy(
        embT_hbm.at[:, pl.ds(_NFULL * _VCH, _VTAIL)], tbuf, tsem).start()

    fcw = fcw_ref[...]
    fcb = fcb_ref[...]

    def _step(c, carry):
        s = lax.rem(c, _NBUF)
        _copy(c, s).wait()
        p = lax.dot_general(fcw, buf[s], (((1,), (0,)), ((), ())),
                            preferred_element_type=jnp.float32)
        out_ref[pl.ds(c * _VCH, _VCH)] = _pack(p, fcb, _VCH)

        @pl.when(c + _NBUF < _NFULL)
        def _():
            _copy(c + _NBUF, s).start()
        return carry

    lax.fori_loop(0, _NFULL, _step, 0)
    pltpu.make_async_copy(
        embT_hbm.at[:, pl.ds(_NFULL * _VCH, _VTAIL)], tbuf, tsem).wait()
    p = lax.dot_general(fcw, tbuf[...], (((1,), (0,)), ((), ())),
                        preferred_element_type=jnp.float32)
    out_ref[pl.ds(_NFULL * _VCH, _VTAIL)] = _pack(p, fcb, _VTAIL)


def _project(embT, fcw, fcb):
    return pl.pallas_call(
        _proj_body,
        in_specs=[
            pl.BlockSpec(memory_space=pl.ANY),
            pl.BlockSpec((_C, _D), lambda: (0, 0)),
            pl.BlockSpec((_C, 1), lambda: (0, 0)),
        ],
        out_specs=pl.BlockSpec((_VOCAB,), lambda: (0,)),
        out_shape=jax.ShapeDtypeStruct((_VOCAB,), jnp.int32),
        scratch_shapes=[
            pltpu.VMEM((_NBUF, _D, _VCH), jnp.float32),
            pltpu.VMEM((_D, _VTAIL), jnp.float32),
            pltpu.SemaphoreType.DMA,
            pltpu.SemaphoreType.DMA((_NBUF,)),
        ],
    )(embT, fcw, fcb)


def _sc_body(p_hbm, idx_hbm, lane_hbm, out_hbm,
             idx_v, rows_v, lane_v, out_v, *sems):
    wid = lax.axis_index("s") * _NC + lax.axis_index("c")
    pltpu.sync_copy(idx_hbm.at[wid], idx_v)
    pltpu.sync_copy(lane_hbm, lane_v)

    lane16 = [lane_v[v] for v in range(8)]
    himask = lane_v[8]
    sh16 = lane_v[9]

    def _gather(j, s):
        return pltpu.make_async_copy(
            p_hbm.at[idx_v.at[j]], rows_v.at[s], sems[s])

    for s in range(_NSLOT):
        _gather(s, s).start()

    def _outer(j0, c):
        for s in range(_NSLOT):
            j = j0 * _NSLOT + s
            _gather(j, s).wait()
            row = rows_v.at[s]
            uv = plsc.load_gather(row, [lane16[0]])
            a0 = plsc.bitcast(uv & himask, jnp.float32)
            a1 = plsc.bitcast(lax.shift_left(uv, sh16), jnp.float32)
            for v in range(1, 8):
                uv = plsc.load_gather(row, [lane16[v]])
                a0 = a0 + plsc.bitcast(uv & himask, jnp.float32)
                a1 = a1 + plsc.bitcast(lax.shift_left(uv, sh16), jnp.float32)
            g = j // _KCH
            k = j - g * _KCH
            o0 = pl.ds(g * 16, 16)
            o1 = pl.ds(128 + g * 16, 16)

            @pl.when(k == 0)
            def _():
                out_v[o0] = a0
                out_v[o1] = a1

            @pl.when(k != 0)
            def _():
                out_v[o0] = out_v[o0] + a0
                out_v[o1] = out_v[o1] + a1

            @pl.when(j + _NSLOT < _NCH)
            def _():
                _gather(j + _NSLOT, s).start()
        return c

    lax.fori_loop(0, _NCH // _NSLOT, _outer, 0)
    pltpu.sync_copy(out_v, out_hbm.at[wid])


_lookup = pl.kernel(
    _sc_body,
    out_type=jax.ShapeDtypeStruct((_NW, _C * _ROWS_W), jnp.float32),
    mesh=plsc.VectorSubcoreMesh(core_axis_name="c", subcore_axis_name="s"),
    compiler_params=pltpu.CompilerParams(
        needs_layout_passes=False, use_tc_tiling_on_sc=False),
    scratch_types=[
        pltpu.VMEM((_NCH, _CH), jnp.int32),
        pltpu.VMEM((_NSLOT, _CH), jnp.int32),
        pltpu.VMEM((10, 16), jnp.int32),
        pltpu.VMEM((_C * _ROWS_W,), jnp.float32),
    ] + [pltpu.SemaphoreType.DMA] * _NSLOT,
)

_LANE_TAB = [[v * 16 + i for i in range(16)] for v in range(8)]
_LANE_TAB.append([-65536] * 16)
_LANE_TAB.append([16] * 16)


def kernel(text, emb_table, fc_w, fc_b):
    embT = emb_table.T
    ptab = _project(embT, fc_w.astype(jnp.float32),
                    fc_b.astype(jnp.float32).reshape(_C, 1))
    textT = text.astype(jnp.int32).T
    u = textT.reshape(_KCH, 8, _NW, _GROUPS, 16)
    idx = u.transpose(2, 3, 0, 1, 4).reshape(_NW, _NCH, _CH)
    lane_tab = jnp.asarray(_LANE_TAB, jnp.int32)
    out32 = _lookup(ptab, idx, lane_tab)
    return out32.reshape(_NW, _C, _ROWS_W).transpose(0, 2, 1).reshape(_B, _C)


# speedup vs baseline: 1.5630x; 1.0231x over previous
import jax
import jax.numpy as jnp
from jax import lax
from jax.experimental import pallas as pl
from jax.experimental.pallas import tpu as pltpu
from jax.experimental.pallas import tpu_sc as plsc

_VOCAB = 100000
_D = 300
_B = 4096
_L = 200
_C = 2

_NC = 2
_NS = 16
_NW = _NC * _NS
_ROWS_W = _B // _NW
_GROUPS = _ROWS_W // 16
_KCH = _L // 8
_NCH = _GROUPS * _KCH
_CH = 128
_NSLOT = 8

_VBLK = 8192


def _proj_body(embT_ref, fcw_ref, fcb_ref, out_ref):
    p = lax.dot_general(fcw_ref[...], embT_ref[...], (((1,), (0,)), ((), ())),
                        preferred_element_type=jnp.float32)
    p = (p + fcb_ref[...]) * (1.0 / _L)
    u = lax.bitcast_convert_type(
        p.astype(jnp.bfloat16), jnp.uint16).astype(jnp.uint32)
    out_ref[...] = lax.bitcast_convert_type((u[0] << 16) | u[1], jnp.int32)


def _project(embT, fcw, fcb):
    return pl.pallas_call(
        _proj_body,
        grid=(pl.cdiv(_VOCAB, _VBLK),),
        in_specs=[
            pl.BlockSpec((_D, _VBLK), lambda i: (0, i)),
            pl.BlockSpec((_C, _D), lambda i: (0, 0)),
            pl.BlockSpec((_C, 1), lambda i: (0, 0)),
        ],
        out_specs=pl.BlockSpec((_VBLK,), lambda i: (i,)),
        out_shape=jax.ShapeDtypeStruct((_VOCAB,), jnp.int32),
    )(embT, fcw, fcb)


def _sc_body(p_hbm, idx_hbm, lane_hbm, out_hbm,
             idx_v, rows_v, lane_v, out_v, *sems):
    wid = lax.axis_index("s") * _NC + lax.axis_index("c")
    pltpu.sync_copy(idx_hbm.at[wid], idx_v)
    pltpu.sync_copy(lane_hbm, lane_v)

    lane16 = [lane_v[v] for v in range(8)]
    himask = lane_v[8]
    sh16 = lane_v[9]

    def _gather(j, s):
        return pltpu.make_async_copy(
            p_hbm.at[idx_v.at[j]], rows_v.at[s], sems[s])

    for s in range(_NSLOT):
        _gather(s, s).start()

    def _outer(j0, c):
        for s in range(_NSLOT):
            j = j0 * _NSLOT + s
            _gather(j, s).wait()
            row = rows_v.at[s]
            uv = plsc.load_gather(row, [lane16[0]])
            a0 = plsc.bitcast(uv & himask, jnp.float32)
            a1 = plsc.bitcast(lax.shift_left(uv, sh16), jnp.float32)
            for v in range(1, 8):
                uv = plsc.load_gather(row, [lane16[v]])
                a0 = a0 + plsc.bitcast(uv & himask, jnp.float32)
                a1 = a1 + plsc.bitcast(lax.shift_left(uv, sh16), jnp.float32)
            g = j // _KCH
            k = j - g * _KCH
            o0 = pl.ds(g * 16, 16)
            o1 = pl.ds(128 + g * 16, 16)

            @pl.when(k == 0)
            def _():
                out_v[o0] = a0
                out_v[o1] = a1

            @pl.when(k != 0)
            def _():
                out_v[o0] = out_v[o0] + a0
                out_v[o1] = out_v[o1] + a1

            @pl.when(j + _NSLOT < _NCH)
            def _():
                _gather(j + _NSLOT, s).start()
        return c

    lax.fori_loop(0, _NCH // _NSLOT, _outer, 0)
    pltpu.sync_copy(out_v, out_hbm.at[wid])


_lookup = pl.kernel(
    _sc_body,
    out_type=jax.ShapeDtypeStruct((_NW, _C * _ROWS_W), jnp.float32),
    mesh=plsc.VectorSubcoreMesh(core_axis_name="c", subcore_axis_name="s"),
    compiler_params=pltpu.CompilerParams(
        needs_layout_passes=False, use_tc_tiling_on_sc=False),
    scratch_types=[
        pltpu.VMEM((_NCH, _CH), jnp.int32),
        pltpu.VMEM((_NSLOT, _CH), jnp.int32),
        pltpu.VMEM((10, 16), jnp.int32),
        pltpu.VMEM((_C * _ROWS_W,), jnp.float32),
    ] + [pltpu.SemaphoreType.DMA] * _NSLOT,
)

_LANE_TAB = [[v * 16 + i for i in range(16)] for v in range(8)]
_LANE_TAB.append([-65536] * 16)
_LANE_TAB.append([16] * 16)


def kernel(text, emb_table, fc_w, fc_b):
    embT = emb_table.T
    ptab = _project(embT, fc_w.astype(jnp.float32),
                    fc_b.astype(jnp.float32).reshape(_C, 1))
    textT = text.astype(jnp.int32).T
    u = textT.reshape(_KCH, 8, _NW, _GROUPS, 16)
    idx = u.transpose(2, 3, 0, 1, 4).reshape(_NW, _NCH, _CH)
    lane_tab = jnp.asarray(_LANE_TAB, jnp.int32)
    out32 = _lookup(ptab, idx, lane_tab)
    return out32.reshape(_NW, _C, _ROWS_W).transpose(0, 2, 1).reshape(_B, _C)
